# baseline (speedup 1.0000x reference)
import functools

import jax
import jax.numpy as jnp
from jax import lax
from jax.experimental import pallas as pl
from jax.experimental.pallas import tpu as pltpu
from jax.experimental.pallas import tpu_sc as plsc

_INT_MAX = 0x7FFFFFFF
_NW = 32


def _mesh():
    return plsc.VectorSubcoreMesh(core_axis_name="c", subcore_axis_name="s")


def _wid():
    return lax.axis_index("s") * 2 + lax.axis_index("c")


def _sc_params():
    return pltpu.CompilerParams(needs_layout_passes=False)


@functools.lru_cache(maxsize=None)
def _winner_call(N, B):
    NR = N // _NW
    CAP_R = (NR + 127) // 128 + 1
    NVREG = B // 16

    @functools.partial(
        pl.kernel,
        out_type=(
            jax.ShapeDtypeStruct((_NW, CAP_R, 128), jnp.int32),
            jax.ShapeDtypeStruct((_NW, CAP_R, 128), jnp.int32),
            jax.ShapeDtypeStruct((_NW, 16), jnp.int32),
        ),
        mesh=_mesh(),
        compiler_params=_sc_params(),
        scratch_types=[
            pltpu.VMEM((B,), jnp.int32),
            pltpu.VMEM((B + 16,), jnp.int32),
            pltpu.VMEM((NR + 16,), jnp.int32),
            pltpu.VMEM((CAP_R, 128), jnp.int32),
            pltpu.VMEM((CAP_R, 128), jnp.int32),
            pltpu.VMEM((34,), jnp.int32),
            pltpu.VMEM((16,), jnp.int32),
        ],
    )
    def win_k(srci, wn_hbm, wev_hbm, wc_hbm, sv, kept, tab, wn2, wev2, tmp, wcv):
        w = _wid()
        lo = w * NR
        iota = lax.iota(jnp.int32, 16)
        imax = jnp.full((16,), _INT_MAX, jnp.int32)
        pltpu.sync_copy(srci, sv)

        def fbody(j, cnt):
            ev = 16 * j + iota
            v = plsc.load_gather(sv, [ev])
            m = (v >= lo) & (v < lo + NR)
            plsc.store_compressed(kept.at[pl.ds(cnt, 16)], v * B + ev, mask=m)
            return cnt + plsc.all_reduce_population_count(m)[0]

        cnt = lax.fori_loop(0, NVREG, fbody, jnp.int32(0))
        plsc.store_scatter(kept, [cnt + iota], imax)
        plsc.store_scatter(tmp, [16 + iota], imax)
        nv = (cnt + 15) // 16

        def dbody(j, carry):
            k = plsc.load_gather(kept, [16 * j + iota])
            srt = lax.sort(k)
            plsc.store_scatter(tmp, [iota], srt)
            nxt = plsc.load_gather(tmp, [iota + 1])
            s_cur = srt >> 14
            keep = ((s_cur != (nxt >> 14)) | (iota == 15)) & (srt != _INT_MAX)
            node = jnp.clip(s_cur - lo, 0, NR - 1)
            plsc.store_scatter(tab, [node], srt & (B - 1), mask=keep)
            return carry

        lax.fori_loop(0, nv, dbody, jnp.int32(0))

        def wbody(j, wcnt):
            k = plsc.load_gather(kept, [16 * j + iota])
            valid = k != _INT_MAX
            s_cur = k >> 14
            node = jnp.clip(s_cur - lo, 0, NR - 1)
            ev = k & (B - 1)
            tv = plsc.load_gather(tab, [node])
            win = valid & (tv == ev)
            wi = win.astype(jnp.int32)
            pos = wcnt + plsc.cumsum(wi) - 1
            r = pos >> 7
            c2 = pos & 127
            plsc.store_scatter(wn2, [r, c2], s_cur, mask=win)
            plsc.store_scatter(wev2, [r, c2], ev, mask=win)
            return wcnt + jnp.sum(wi)

        wcnt = lax.fori_loop(0, nv, wbody, jnp.int32(0))

        nrows = (wcnt + 127) >> 7
        rowi = jnp.maximum(nrows - 1, 0)
        rsplat = jnp.full((16,), rowi, jnp.int32)
        zeros = jnp.zeros((16,), jnp.int32)
        v0n = plsc.load_gather(wn2, [rsplat, zeros])
        v0e = plsc.load_gather(wev2, [rsplat, zeros])
        rem = wcnt & 127
        rem2 = jnp.where(rem == 0, 128, rem)
        for kk in range(8):
            cpos = 16 * kk + iota
            mkeep = cpos < rem2
            curn = plsc.load_gather(wn2, [rsplat, cpos])
            cure = plsc.load_gather(wev2, [rsplat, cpos])
            plsc.store_scatter(wn2, [rsplat, cpos], jnp.where(mkeep, curn, v0n))
            plsc.store_scatter(wev2, [rsplat, cpos], jnp.where(mkeep, cure, v0e))

        plsc.store_scatter(wcv, [iota], jnp.full((16,), wcnt, jnp.int32))
        pltpu.sync_copy(wn2, wn_hbm.at[w])
        pltpu.sync_copy(wev2, wev_hbm.at[w])
        pltpu.sync_copy(wcv, wc_hbm.at[w])

    return win_k


@functools.lru_cache(maxsize=None)
def _gather_call(N, B, D):
    EV = B // _NW
    CH = 128
    NCH = EV // CH

    @functools.partial(
        pl.kernel,
        out_type=(
            jax.ShapeDtypeStruct((B, D), jnp.float32),
            jax.ShapeDtypeStruct((B, D), jnp.float32),
            jax.ShapeDtypeStruct((B,), jnp.float32),
        ),
        mesh=_mesh(),
        compiler_params=_sc_params(),
        scratch_types=[
            pltpu.VMEM((NCH, CH), jnp.int32),
            pltpu.VMEM((NCH, CH), jnp.int32),
            pltpu.VMEM((EV,), jnp.int32),
            pltpu.VMEM((CH, D), jnp.float32),
            pltpu.VMEM((CH, D), jnp.float32),
            pltpu.VMEM((CH, D), jnp.float32),
            pltpu.VMEM((CH, D), jnp.float32),
            pltpu.VMEM((EV,), jnp.float32),
            pltpu.SemaphoreType.DMA,
            pltpu.SemaphoreType.DMA,
            pltpu.SemaphoreType.DMA,
            pltpu.SemaphoreType.DMA,
            pltpu.SemaphoreType.DMA,
        ],
    )
    def gather_k(mem, lu, srci, dsti, hsrc, hdst, lug,
                 sidx2, didx2, sflat, ra0, ra1, rb0, rb1, luv,
                 sem_ga, sem_gb, sem_wa, sem_wb, sem_c):
        w = _wid()
        base = w * EV
        for c in range(NCH):
            pltpu.sync_copy(srci.at[pl.ds(base + c * CH, CH)], sidx2.at[c])
            pltpu.sync_copy(dsti.at[pl.ds(base + c * CH, CH)], didx2.at[c])
        pltpu.sync_copy(srci.at[pl.ds(base, EV)], sflat)
        lu_dma = pltpu.async_copy(lu.at[sflat], luv, sem_c)

        ras = [ra0, ra1]
        rbs = [rb0, rb1]
        pltpu.async_copy(mem.at[sidx2.at[0]], ra0, sem_ga)
        pltpu.async_copy(mem.at[didx2.at[0]], rb0, sem_gb)
        for c in range(NCH):
            ra_cur, ra_nxt = ras[c % 2], ras[(c + 1) % 2]
            rb_cur, rb_nxt = rbs[c % 2], rbs[(c + 1) % 2]
            pltpu.make_async_copy(mem.at[sidx2.at[c]], ra_cur, sem_ga).wait()
            pltpu.make_async_copy(mem.at[didx2.at[c]], rb_cur, sem_gb).wait()
            if c + 1 < NCH:
                if c >= 1:
                    pltpu.make_async_copy(ra_nxt, hsrc.at[pl.ds(0, CH)], sem_wa).wait()
                    pltpu.make_async_copy(rb_nxt, hdst.at[pl.ds(0, CH)], sem_wb).wait()
                pltpu.async_copy(mem.at[sidx2.at[c + 1]], ra_nxt, sem_ga)
                pltpu.async_copy(mem.at[didx2.at[c + 1]], rb_nxt, sem_gb)
            pltpu.async_copy(ra_cur, hsrc.at[pl.ds(base + c * CH, CH)], sem_wa)
            pltpu.async_copy(rb_cur, hdst.at[pl.ds(base + c * CH, CH)], sem_wb)
        for _ in range(2 if NCH >= 2 else 1):
            pltpu.make_async_copy(ra0, hsrc.at[pl.ds(0, CH)], sem_wa).wait()
            pltpu.make_async_copy(rb0, hdst.at[pl.ds(0, CH)], sem_wb).wait()
        lu_dma.wait()
        pltpu.sync_copy(luv, lug.at[pl.ds(base, EV)])

    return gather_k


@functools.lru_cache(maxsize=None)
def _gru_call(B, D, E, T, N):
    BLK = 512
    G = B // BLK
    CPB = 2000
    GC = N // CPB
    assert GC >= G
    M3 = 3 * D

    def body(hs_ref, hd_ref, ef_ref, ts_ref, lu_ref, tw_ref, tb_ref,
             wih_ref, whh_ref, bih_ref, bhh_ref, mem_ref, out_ref, cp_ref):
        cp_ref[...] = mem_ref[...]
        hs = hs_ref[...]
        hd = hd_ref[...]
        ef = ef_ref[...]
        dt = ts_ref[...] - lu_ref[...]
        x = dt * tw_ref[...] + tb_ref[...]
        k = jnp.round(x * jnp.float32(0.15915494309189535))
        r = x - k * jnp.float32(6.28125)
        r = r - k * jnp.float32(0.0019353071795864769)
        u = r * r
        tenc = jnp.float32(1.711475536281e-09)
        for cc in (-2.704132919043e-07, 2.476580580219e-05,
                   -1.388760105134e-03, 4.166644395024e-02,
                   -4.999998542388e-01, 9.999999843295e-01):
            tenc = tenc * u + jnp.float32(cc)
        bf16 = jnp.bfloat16
        wih = wih_ref[...].astype(bf16)
        whh = whh_ref[...].astype(bf16)
        hsb = hs.astype(bf16)
        dgn = (((1,), (1,)), ((), ()))
        f32 = jnp.float32
        gx = (lax.dot_general(hsb, wih[:, 0:D], dgn, preferred_element_type=f32)
              + lax.dot_general(hd.astype(bf16), wih[:, D:2 * D], dgn, preferred_element_type=f32)
              + lax.dot_general(ef.astype(bf16), wih[:, 2 * D:2 * D + E], dgn, preferred_element_type=f32)
              + lax.dot_general(tenc.astype(bf16), wih[:, 2 * D + E:], dgn, preferred_element_type=f32)
              + bih_ref[...])
        gh = lax.dot_general(hsb, whh, dgn, preferred_element_type=f32) + bhh_ref[...]
        r = 1.0 / (1.0 + jnp.exp(-(gx[:, 0:D] + gh[:, 0:D])))
        z = 1.0 / (1.0 + jnp.exp(-(gx[:, D:2 * D] + gh[:, D:2 * D])))
        n = jnp.tanh(gx[:, 2 * D:] + r * gh[:, 2 * D:])
        out_ref[...] = (1.0 - z) * n + z * hs

    clamp = G - 1

    def evmap(i):
        return (jnp.minimum(i, clamp), 0)

    return pl.pallas_call(
        body,
        grid=(GC,),
        in_specs=[
            pl.BlockSpec((BLK, D), evmap),
            pl.BlockSpec((BLK, D), evmap),
            pl.BlockSpec((BLK, E), evmap),
            pl.BlockSpec((BLK, 1), evmap),
            pl.BlockSpec((BLK, 1), evmap),
            pl.BlockSpec((1, T), lambda i: (0, 0)),
            pl.BlockSpec((1, T), lambda i: (0, 0)),
            pl.BlockSpec((M3, 2 * D + E + T), lambda i: (0, 0)),
            pl.BlockSpec((M3, D), lambda i: (0, 0)),
            pl.BlockSpec((1, M3), lambda i: (0, 0)),
            pl.BlockSpec((1, M3), lambda i: (0, 0)),
            pl.BlockSpec((CPB, D), lambda i: (i, 0)),
        ],
        out_specs=(pl.BlockSpec((BLK, D), evmap),
                   pl.BlockSpec((CPB, D), lambda i: (i, 0))),
        out_shape=(jax.ShapeDtypeStruct((B, D), jnp.float32),
                   jax.ShapeDtypeStruct((N, D), jnp.float32)),
    )


@functools.lru_cache(maxsize=None)
def _scatter_call(N, B, D):
    NR = N // _NW
    CAP_R = (NR + 127) // 128 + 1

    @functools.partial(
        pl.kernel,
        out_type=(),
        mesh=_mesh(),
        compiler_params=_sc_params(),
        scratch_types=[
            pltpu.VMEM((CAP_R, 128), jnp.int32),
            pltpu.VMEM((CAP_R, 128), jnp.int32),
            pltpu.VMEM((16,), jnp.int32),
            pltpu.VMEM((128, D), jnp.float32),
            pltpu.VMEM((128, D), jnp.float32),
            pltpu.SemaphoreType.DMA,
        ],
    )
    def scat_k(newh, wn_hbm, wev_hbm, wc_hbm, out, wn2, wev2, wcv, r0, r1, sem_g):
        w = _wid()
        pltpu.sync_copy(wn_hbm.at[w], wn2)
        pltpu.sync_copy(wev_hbm.at[w], wev2)
        pltpu.sync_copy(wc_hbm.at[w], wcv)
        wcnt = jnp.max(wcv[...])
        nrows = (wcnt + 127) >> 7

        bufs = [r0, r1]

        @pl.when(jnp.int32(0) < nrows)
        def _():
            pltpu.async_copy(newh.at[wev2.at[0]], r0, sem_g)

        for c in range(CAP_R):
            buf, nbuf = bufs[c % 2], bufs[(c + 1) % 2]

            @pl.when(jnp.int32(c) < nrows)
            def _():
                pltpu.make_async_copy(newh.at[wev2.at[c]], buf, sem_g).wait()

            if c + 1 < CAP_R:
                @pl.when(jnp.int32(c + 1) < nrows)
                def _():
                    pltpu.async_copy(newh.at[wev2.at[c + 1]], nbuf, sem_g)

            @pl.when(jnp.int32(c) < nrows)
            def _():
                pltpu.sync_copy(buf, out.at[wn2.at[c]])

    return scat_k


def kernel(memory, last_update, edge_feat, timestamps, time_w, time_b,
           W_ih, W_hh, b_ih, b_hh, src_idx, dst_idx):
    N, D = memory.shape
    B = src_idx.shape[0]
    E = edge_feat.shape[1]
    T = time_w.shape[0]
    src = src_idx.astype(jnp.int32)
    dst = dst_idx.astype(jnp.int32)

    wn, wev, wc = _winner_call(N, B)(src)
    hsrc, hdst, lug = _gather_call(N, B, D)(memory, last_update, src, dst)
    newh, mcopy = _gru_call(B, D, E, T, N)(
        hsrc, hdst, edge_feat,
        timestamps.reshape(B, 1), lug.reshape(B, 1),
        time_w.reshape(1, T), time_b.reshape(1, T),
        W_ih, W_hh, b_ih.reshape(1, 3 * D), b_hh.reshape(1, 3 * D), memory)

    out_ref = jax.new_ref(mcopy)
    _scatter_call(N, B, D)(newh, wn, wev, wc, out_ref)
    return out_ref[...]


# speedup vs baseline: 1.0489x; 1.0489x over previous
import functools

import jax
import jax.numpy as jnp
from jax import lax
from jax.experimental import pallas as pl
from jax.experimental.pallas import tpu as pltpu
from jax.experimental.pallas import tpu_sc as plsc

_INT_MAX = 0x7FFFFFFF
_NW = 32


def _mesh():
    return plsc.VectorSubcoreMesh(core_axis_name="c", subcore_axis_name="s")


def _wid():
    return lax.axis_index("s") * 2 + lax.axis_index("c")


def _sc_params():
    return pltpu.CompilerParams(needs_layout_passes=False)


@functools.lru_cache(maxsize=None)
def _winner_call(N, B):
    NR = N // _NW
    CAP_R = (NR + 127) // 128 + 1
    NVREG = B // 16

    @functools.partial(
        pl.kernel,
        out_type=(
            jax.ShapeDtypeStruct((_NW, CAP_R, 128), jnp.int32),
            jax.ShapeDtypeStruct((_NW, CAP_R, 128), jnp.int32),
            jax.ShapeDtypeStruct((_NW, 16), jnp.int32),
        ),
        mesh=_mesh(),
        compiler_params=_sc_params(),
        scratch_types=[
            pltpu.VMEM((B,), jnp.int32),
            pltpu.VMEM((B + 16,), jnp.int32),
            pltpu.VMEM((NR + 16,), jnp.int32),
            pltpu.VMEM((CAP_R, 128), jnp.int32),
            pltpu.VMEM((CAP_R, 128), jnp.int32),
            pltpu.VMEM((34,), jnp.int32),
            pltpu.VMEM((16,), jnp.int32),
        ],
    )
    def win_k(srci, wn_hbm, wev_hbm, wc_hbm, sv, kept, tab, wn2, wev2, tmp, wcv):
        w = _wid()
        lo = w * NR
        iota = lax.iota(jnp.int32, 16)
        imax = jnp.full((16,), _INT_MAX, jnp.int32)
        pltpu.sync_copy(srci, sv)

        def fbody(j, cnt):
            ev = 16 * j + iota
            v = plsc.load_gather(sv, [ev])
            m = (v >= lo) & (v < lo + NR)
            plsc.store_compressed(kept.at[pl.ds(cnt, 16)], v * B + ev, mask=m)
            return cnt + plsc.all_reduce_population_count(m)[0]

        cnt = lax.fori_loop(0, NVREG, fbody, jnp.int32(0))
        plsc.store_scatter(kept, [cnt + iota], imax)
        plsc.store_scatter(tmp, [16 + iota], imax)
        nv = (cnt + 15) // 16

        def dbody(j, carry):
            k = plsc.load_gather(kept, [16 * j + iota])
            srt = lax.sort(k)
            plsc.store_scatter(tmp, [iota], srt)
            nxt = plsc.load_gather(tmp, [iota + 1])
            s_cur = srt >> 14
            keep = ((s_cur != (nxt >> 14)) | (iota == 15)) & (srt != _INT_MAX)
            node = jnp.clip(s_cur - lo, 0, NR - 1)
            plsc.store_scatter(tab, [node], srt & (B - 1), mask=keep)
            return carry

        lax.fori_loop(0, nv, dbody, jnp.int32(0))

        def wbody(j, wcnt):
            k = plsc.load_gather(kept, [16 * j + iota])
            valid = k != _INT_MAX
            s_cur = k >> 14
            node = jnp.clip(s_cur - lo, 0, NR - 1)
            ev = k & (B - 1)
            tv = plsc.load_gather(tab, [node])
            win = valid & (tv == ev)
            wi = win.astype(jnp.int32)
            pos = wcnt + plsc.cumsum(wi) - 1
            r = pos >> 7
            c2 = pos & 127
            plsc.store_scatter(wn2, [r, c2], s_cur, mask=win)
            plsc.store_scatter(wev2, [r, c2], ev, mask=win)
            return wcnt + jnp.sum(wi)

        wcnt = lax.fori_loop(0, nv, wbody, jnp.int32(0))

        nrows = (wcnt + 127) >> 7
        rowi = jnp.maximum(nrows - 1, 0)
        rsplat = jnp.full((16,), rowi, jnp.int32)
        zeros = jnp.zeros((16,), jnp.int32)
        v0n = plsc.load_gather(wn2, [rsplat, zeros])
        v0e = plsc.load_gather(wev2, [rsplat, zeros])
        rem = wcnt & 127
        rem2 = jnp.where(rem == 0, 128, rem)
        for kk in range(8):
            cpos = 16 * kk + iota
            mkeep = cpos < rem2
            curn = plsc.load_gather(wn2, [rsplat, cpos])
            cure = plsc.load_gather(wev2, [rsplat, cpos])
            plsc.store_scatter(wn2, [rsplat, cpos], jnp.where(mkeep, curn, v0n))
            plsc.store_scatter(wev2, [rsplat, cpos], jnp.where(mkeep, cure, v0e))

        plsc.store_scatter(wcv, [iota], jnp.full((16,), wcnt, jnp.int32))
        pltpu.sync_copy(wn2, wn_hbm.at[w])
        pltpu.sync_copy(wev2, wev_hbm.at[w])
        pltpu.sync_copy(wcv, wc_hbm.at[w])

    return win_k


@functools.lru_cache(maxsize=None)
def _gather_call(N, B, D):
    EV = B // _NW
    CH = 128
    NCH = EV // CH

    @functools.partial(
        pl.kernel,
        out_type=(
            jax.ShapeDtypeStruct((B, D), jnp.float32),
            jax.ShapeDtypeStruct((B, D), jnp.float32),
            jax.ShapeDtypeStruct((B,), jnp.float32),
        ),
        mesh=_mesh(),
        compiler_params=_sc_params(),
        scratch_types=[
            pltpu.VMEM((NCH, CH), jnp.int32),
            pltpu.VMEM((NCH, CH), jnp.int32),
            pltpu.VMEM((EV,), jnp.int32),
            pltpu.VMEM((CH, D), jnp.float32),
            pltpu.VMEM((CH, D), jnp.float32),
            pltpu.VMEM((CH, D), jnp.float32),
            pltpu.VMEM((CH, D), jnp.float32),
            pltpu.VMEM((EV,), jnp.float32),
            pltpu.SemaphoreType.DMA,
            pltpu.SemaphoreType.DMA,
            pltpu.SemaphoreType.DMA,
            pltpu.SemaphoreType.DMA,
            pltpu.SemaphoreType.DMA,
        ],
    )
    def gather_k(mem, lu, srci, dsti, hsrc, hdst, lug,
                 sidx2, didx2, sflat, ra0, ra1, rb0, rb1, luv,
                 sem_ga, sem_gb, sem_wa, sem_wb, sem_c):
        w = _wid()
        base = w * EV
        for c in range(NCH):
            pltpu.sync_copy(srci.at[pl.ds(base + c * CH, CH)], sidx2.at[c])
            pltpu.sync_copy(dsti.at[pl.ds(base + c * CH, CH)], didx2.at[c])
        pltpu.sync_copy(srci.at[pl.ds(base, EV)], sflat)
        lu_dma = pltpu.async_copy(lu.at[sflat], luv, sem_c)

        ras = [ra0, ra1]
        rbs = [rb0, rb1]
        pltpu.async_copy(mem.at[sidx2.at[0]], ra0, sem_ga)
        pltpu.async_copy(mem.at[didx2.at[0]], rb0, sem_gb)
        for c in range(NCH):
            ra_cur, ra_nxt = ras[c % 2], ras[(c + 1) % 2]
            rb_cur, rb_nxt = rbs[c % 2], rbs[(c + 1) % 2]
            pltpu.make_async_copy(mem.at[sidx2.at[c]], ra_cur, sem_ga).wait()
            pltpu.make_async_copy(mem.at[didx2.at[c]], rb_cur, sem_gb).wait()
            if c + 1 < NCH:
                if c >= 1:
                    pltpu.make_async_copy(ra_nxt, hsrc.at[pl.ds(0, CH)], sem_wa).wait()
                    pltpu.make_async_copy(rb_nxt, hdst.at[pl.ds(0, CH)], sem_wb).wait()
                pltpu.async_copy(mem.at[sidx2.at[c + 1]], ra_nxt, sem_ga)
                pltpu.async_copy(mem.at[didx2.at[c + 1]], rb_nxt, sem_gb)
            pltpu.async_copy(ra_cur, hsrc.at[pl.ds(base + c * CH, CH)], sem_wa)
            pltpu.async_copy(rb_cur, hdst.at[pl.ds(base + c * CH, CH)], sem_wb)
        for _ in range(2 if NCH >= 2 else 1):
            pltpu.make_async_copy(ra0, hsrc.at[pl.ds(0, CH)], sem_wa).wait()
            pltpu.make_async_copy(rb0, hdst.at[pl.ds(0, CH)], sem_wb).wait()
        lu_dma.wait()
        pltpu.sync_copy(luv, lug.at[pl.ds(base, EV)])

    return gather_k


@functools.lru_cache(maxsize=None)
def _gru_call(B, D, E, T):
    BLK = 512
    G = B // BLK
    M3 = 3 * D

    def body(hs_ref, hd_ref, ef_ref, ts_ref, lu_ref, tw_ref, tb_ref,
             wih_ref, whh_ref, bih_ref, bhh_ref, out_ref):
        hs = hs_ref[...]
        hd = hd_ref[...]
        ef = ef_ref[...]
        dt = ts_ref[...] - lu_ref[...]
        x = dt * tw_ref[...] + tb_ref[...]
        k = jnp.round(x * jnp.float32(0.15915494309189535))
        r = x - k * jnp.float32(6.28125)
        r = r - k * jnp.float32(0.0019353071795864769)
        u = r * r
        tenc = jnp.float32(1.711475536281e-09)
        for cc in (-2.704132919043e-07, 2.476580580219e-05,
                   -1.388760105134e-03, 4.166644395024e-02,
                   -4.999998542388e-01, 9.999999843295e-01):
            tenc = tenc * u + jnp.float32(cc)
        bf16 = jnp.bfloat16
        wih = wih_ref[...].astype(bf16)
        whh = whh_ref[...].astype(bf16)
        hsb = hs.astype(bf16)
        dgn = (((1,), (1,)), ((), ()))
        f32 = jnp.float32
        gx = (lax.dot_general(hsb, wih[:, 0:D], dgn, preferred_element_type=f32)
              + lax.dot_general(hd.astype(bf16), wih[:, D:2 * D], dgn, preferred_element_type=f32)
              + lax.dot_general(ef.astype(bf16), wih[:, 2 * D:2 * D + E], dgn, preferred_element_type=f32)
              + lax.dot_general(tenc.astype(bf16), wih[:, 2 * D + E:], dgn, preferred_element_type=f32)
              + bih_ref[...])
        gh = lax.dot_general(hsb, whh, dgn, preferred_element_type=f32) + bhh_ref[...]
        r = 1.0 / (1.0 + jnp.exp(-(gx[:, 0:D] + gh[:, 0:D])))
        z = 1.0 / (1.0 + jnp.exp(-(gx[:, D:2 * D] + gh[:, D:2 * D])))
        n = jnp.tanh(gx[:, 2 * D:] + r * gh[:, 2 * D:])
        out_ref[...] = (1.0 - z) * n + z * hs

    return pl.pallas_call(
        body,
        grid=(G,),
        in_specs=[
            pl.BlockSpec((BLK, D), lambda i: (i, 0)),
            pl.BlockSpec((BLK, D), lambda i: (i, 0)),
            pl.BlockSpec((BLK, E), lambda i: (i, 0)),
            pl.BlockSpec((BLK, 1), lambda i: (i, 0)),
            pl.BlockSpec((BLK, 1), lambda i: (i, 0)),
            pl.BlockSpec((1, T), lambda i: (0, 0)),
            pl.BlockSpec((1, T), lambda i: (0, 0)),
            pl.BlockSpec((M3, 2 * D + E + T), lambda i: (0, 0)),
            pl.BlockSpec((M3, D), lambda i: (0, 0)),
            pl.BlockSpec((1, M3), lambda i: (0, 0)),
            pl.BlockSpec((1, M3), lambda i: (0, 0)),
        ],
        out_specs=pl.BlockSpec((BLK, D), lambda i: (i, 0)),
        out_shape=jax.ShapeDtypeStruct((B, D), jnp.float32),
    )


@functools.lru_cache(maxsize=None)
def _scatter_call(N, B, D):
    NR = N // _NW
    CAP_R = (NR + 127) // 128 + 1

    @functools.partial(
        pl.kernel,
        out_type=(),
        mesh=_mesh(),
        compiler_params=_sc_params(),
        scratch_types=[
            pltpu.VMEM((CAP_R, 128), jnp.int32),
            pltpu.VMEM((CAP_R, 128), jnp.int32),
            pltpu.VMEM((16,), jnp.int32),
            pltpu.VMEM((128, D), jnp.float32),
            pltpu.VMEM((128, D), jnp.float32),
            pltpu.SemaphoreType.DMA,
        ],
    )
    def scat_k(newh, wn_hbm, wev_hbm, wc_hbm, out, wn2, wev2, wcv, r0, r1, sem_g):
        w = _wid()
        pltpu.sync_copy(wn_hbm.at[w], wn2)
        pltpu.sync_copy(wev_hbm.at[w], wev2)
        pltpu.sync_copy(wc_hbm.at[w], wcv)
        wcnt = jnp.max(wcv[...])
        nrows = (wcnt + 127) >> 7

        bufs = [r0, r1]

        @pl.when(jnp.int32(0) < nrows)
        def _():
            pltpu.async_copy(newh.at[wev2.at[0]], r0, sem_g)

        for c in range(CAP_R):
            buf, nbuf = bufs[c % 2], bufs[(c + 1) % 2]

            @pl.when(jnp.int32(c) < nrows)
            def _():
                pltpu.make_async_copy(newh.at[wev2.at[c]], buf, sem_g).wait()

            if c + 1 < CAP_R:
                @pl.when(jnp.int32(c + 1) < nrows)
                def _():
                    pltpu.async_copy(newh.at[wev2.at[c + 1]], nbuf, sem_g)

            @pl.when(jnp.int32(c) < nrows)
            def _():
                pltpu.sync_copy(buf, out.at[wn2.at[c]])

    return scat_k


def kernel(memory, last_update, edge_feat, timestamps, time_w, time_b,
           W_ih, W_hh, b_ih, b_hh, src_idx, dst_idx):
    N, D = memory.shape
    B = src_idx.shape[0]
    E = edge_feat.shape[1]
    T = time_w.shape[0]
    src = src_idx.astype(jnp.int32)
    dst = dst_idx.astype(jnp.int32)

    wn, wev, wc = _winner_call(N, B)(src)
    hsrc, hdst, lug = _gather_call(N, B, D)(memory, last_update, src, dst)
    newh = _gru_call(B, D, E, T)(
        hsrc, hdst, edge_feat,
        timestamps.reshape(B, 1), lug.reshape(B, 1),
        time_w.reshape(1, T), time_b.reshape(1, T),
        W_ih, W_hh, b_ih.reshape(1, 3 * D), b_hh.reshape(1, 3 * D))

    out_ref = jax.new_ref(memory)
    _scatter_call(N, B, D)(newh, wn, wev, wc, out_ref)
    return out_ref[...]
